# baseline (speedup 1.0000x reference)
import functools

import jax
import jax.numpy as jnp
from jax import lax
from jax.experimental import pallas as pl
from jax.experimental.pallas import tpu as pltpu
from jax.experimental.pallas import tpu_sc as plsc

_B, _H, _W, _P = 16, 512, 512, 20000
_NW = 32
_NPT = (_B * _P) // _NW
_CHUNK = 32
_NCH = 314
_SPLIT = 157
_SVEC = _SPLIT * _CHUNK // 16
_APAD = _NCH * _CHUNK
_NPAD = (_APAD - _NPT) // 16
_NVEC = _NPT // 16
_LN2 = 0.6931471805599453


def _softplus_ln(y):
    bits = lax.bitcast_convert_type(y, jnp.int32)
    k = (bits >> 23) - 127
    m = lax.bitcast_convert_type((bits & 0x007FFFFF) | 0x3F800000, jnp.float32)
    t = (m - 1.0) / (m + 1.0)
    s = t * t
    ln_m = 2.0 * t * (1.0 + s * (1.0 / 3.0 + s * (1.0 / 5.0 + s * (1.0 / 7.0))))
    return k.astype(jnp.float32) * _LN2 + ln_m


def _sc_body(img, xa, ya, xb, yb, tt, parts,
             xa_v, ya_v, xb_v, yb_v, t_v, idx_v, z_v, out_v, sem, sem_t,
             sem_c):
    wid = lax.axis_index("s") * 2 + lax.axis_index("c")
    base = wid * _NPT
    boff = (wid // 2) * (_H * _W)

    stages = [
        pltpu.make_async_copy(src.at[pl.ds(base, _NPT)],
                              dst.at[pl.ds(0, _NPT)], sem)
        for src, dst in ((xa, xa_v), (ya, ya_v), (xb, xb_v), (yb, yb_v))
    ]
    for h in stages:
        h.start()
    for h in stages:
        h.wait()

    zeros = jnp.zeros((16,), jnp.int32)
    for j in range(_NPAD):
        for ref in (xa_v, ya_v, xb_v, yb_v):
            ref[pl.ds(_NPT + j * 16, 16)] = zeros

    def make_build_fire(gsem):
      def build_fire(j, _):
        off = pl.multiple_of(j * _CHUNK, _CHUNK)
        for s in range(_CHUNK // 16):
            o = off + s * 16
            xav = xa_v[pl.ds(o, 16)]
            yav = ya_v[pl.ds(o, 16)]
            xbv = xb_v[pl.ds(o, 16)]
            ybv = yb_v[pl.ds(o, 16)]
            idx_v[pl.ds(o, 16)] = (
                (xav << 9) - (xav & 7) * 384
                + (yav << 3) - (yav & 127) * 7 + boff)
            idx_v[pl.ds(_APAD + o, 16)] = (
                (xbv << 9) - (xbv & 7) * 384
                + (ybv << 3) - (ybv & 127) * 7 + boff)
        pltpu.make_async_copy(
            img.at[idx_v.at[pl.ds(off, _CHUNK)]],
            z_v.at[pl.ds(off, _CHUNK)], gsem).start()
        pltpu.make_async_copy(
            img.at[idx_v.at[pl.ds(_APAD + off, _CHUNK)]],
            z_v.at[pl.ds(_APAD + off, _CHUNK)], gsem).start()
        return 0

      return build_fire

    lax.fori_loop(0, _SPLIT, make_build_fire(sem), 0, unroll=2)
    lax.fori_loop(_SPLIT, _NCH, make_build_fire(sem_c), 0, unroll=2)

    t_stage = pltpu.make_async_copy(tt.at[pl.ds(base, _NPT)],
                                    t_v.at[pl.ds(0, _NPT)], sem_t)
    t_stage.start()

    def loss(i, acc):
        acc_log, acc_sq, acc_cnt = acc
        off = pl.multiple_of(i * 16, 16)
        za = z_v[pl.ds(off, 16)]
        zb = z_v[pl.ds(_APAD + off, 16)]
        tf = t_v[pl.ds(off, 16)].astype(jnp.float32)
        pred = za - zb
        e = jnp.exp(-tf * pred)
        ln_y = _softplus_ln(1.0 + e)
        nzf = tf * tf
        p2 = pred * pred
        acc_log = acc_log + nzf * ln_y
        acc_sq = acc_sq + (p2 - nzf * p2)
        acc_cnt = acc_cnt + nzf
        return acc_log, acc_sq, acc_cnt

    z16 = jnp.zeros((16,), jnp.float32)
    half = _SPLIT * _CHUNK

    pltpu.make_async_copy(img.at[pl.ds(0, half)],
                          z_v.at[pl.ds(0, half)], sem).wait()
    pltpu.make_async_copy(img.at[pl.ds(0, half)],
                          z_v.at[pl.ds(_APAD, half)], sem).wait()
    t_stage.wait()
    accs = lax.fori_loop(0, _SVEC, loss, (z16, z16, z16), unroll=8)

    rest = _APAD - half
    pltpu.make_async_copy(img.at[pl.ds(0, rest)],
                          z_v.at[pl.ds(half, rest)], sem_c).wait()
    pltpu.make_async_copy(img.at[pl.ds(0, rest)],
                          z_v.at[pl.ds(_APAD + half, rest)], sem_c).wait()
    acc_log, acc_sq, acc_cnt = lax.fori_loop(_SVEC, _NVEC, loss, accs,
                                             unroll=8)

    out_v[pl.ds(0, 16)] = acc_log
    out_v[pl.ds(16, 16)] = acc_sq
    out_v[pl.ds(32, 16)] = acc_cnt
    pltpu.sync_copy(out_v, parts.at[wid])


@jax.jit
def kernel(output, x_A, y_A, x_B, y_B, ordinal_relation):
    img = (output.reshape(_B, _H // 8, 8, _W // 128, 128)
           .transpose(0, 1, 3, 2, 4).reshape(-1))
    mesh = plsc.VectorSubcoreMesh(core_axis_name="c", subcore_axis_name="s")
    run = functools.partial(
        pl.kernel,
        mesh=mesh,
        out_type=jax.ShapeDtypeStruct((_NW, 48), jnp.float32),
        scratch_types=[
            pltpu.VMEM((_APAD,), jnp.int32),
            pltpu.VMEM((_APAD,), jnp.int32),
            pltpu.VMEM((_APAD,), jnp.int32),
            pltpu.VMEM((_APAD,), jnp.int32),
            pltpu.VMEM((_APAD,), jnp.int32),
            pltpu.VMEM((2 * _APAD,), jnp.int32),
            pltpu.VMEM((2 * _APAD,), jnp.float32),
            pltpu.VMEM((48,), jnp.float32),
            pltpu.SemaphoreType.DMA,
            pltpu.SemaphoreType.DMA,
            pltpu.SemaphoreType.DMA,
        ],
    )(_sc_body)
    parts = run(
        img,
        x_A.reshape(-1).astype(jnp.int32),
        y_A.reshape(-1).astype(jnp.int32),
        x_B.reshape(-1).astype(jnp.int32),
        y_B.reshape(-1).astype(jnp.int32),
        ordinal_relation.reshape(-1).astype(jnp.int32),
    )
    p = parts.reshape(_NW, 3, 16).sum(axis=2).reshape(_B, 2, 3).sum(axis=1)
    cnt_nz = p[:, 2]
    log_loss = p[:, 0] / jnp.maximum(cnt_nz, 1.0)
    sq_loss = p[:, 1] / jnp.maximum(_P - cnt_nz, 1.0)
    return jnp.sum(log_loss + sq_loss) / _B


# speedup vs baseline: 1.0066x; 1.0066x over previous
import functools

import jax
import jax.numpy as jnp
from jax import lax
from jax.experimental import pallas as pl
from jax.experimental.pallas import tpu as pltpu
from jax.experimental.pallas import tpu_sc as plsc

_B, _H, _W, _P = 16, 512, 512, 20000
_NW = 32
_NPT = (_B * _P) // _NW
_CHUNK = 16
_NCH = 625
_APAD = _NCH * _CHUNK
_NPAD = (_APAD - _NPT) // 16
_NVEC = _NPT // 16
_LN2 = 0.6931471805599453


def _softplus_ln(y):
    bits = lax.bitcast_convert_type(y, jnp.int32)
    k = (bits >> 23) - 127
    m = lax.bitcast_convert_type((bits & 0x007FFFFF) | 0x3F800000, jnp.float32)
    t = (m - 1.0) / (m + 1.0)
    s = t * t
    ln_m = 2.0 * t * (1.0 + s * (1.0 / 3.0 + s * (1.0 / 5.0 + s * (1.0 / 7.0))))
    return k.astype(jnp.float32) * _LN2 + ln_m


def _sc_body(img, xa, ya, xb, yb, tt, parts,
             xa_v, ya_v, xb_v, yb_v, t_v, idx_v, z_v, out_v, sem, sem_t):
    wid = lax.axis_index("s") * 2 + lax.axis_index("c")
    base = wid * _NPT
    boff = (wid // 2) * (_H * _W)

    stages = [
        pltpu.make_async_copy(src.at[pl.ds(base, _NPT)],
                              dst.at[pl.ds(0, _NPT)], sem)
        for src, dst in ((xa, xa_v), (ya, ya_v), (xb, xb_v), (yb, yb_v))
    ]
    for h in stages:
        h.start()
    for h in stages:
        h.wait()

    zeros = jnp.zeros((16,), jnp.int32)
    for j in range(_NPAD):
        for ref in (xa_v, ya_v, xb_v, yb_v):
            ref[pl.ds(_NPT + j * 16, 16)] = zeros

    def build_fire(j, _):
        off = pl.multiple_of(j * _CHUNK, _CHUNK)
        for s in range(_CHUNK // 16):
            o = off + s * 16
            xav = xa_v[pl.ds(o, 16)]
            yav = ya_v[pl.ds(o, 16)]
            xbv = xb_v[pl.ds(o, 16)]
            ybv = yb_v[pl.ds(o, 16)]
            iva = ((xav << 9) - (xav & 7) * 384
                   + (yav << 3) - (yav & 127) * 7 + boff)
            ivb = ((xbv << 9) - (xbv & 7) * 384
                   + (ybv << 3) - (ybv & 127) * 7 + boff)
            pltpu.make_async_copy(
                img.at[iva], z_v.at[pl.ds(o, 16)], sem).start()
            pltpu.make_async_copy(
                img.at[ivb], z_v.at[pl.ds(_APAD + o, 16)], sem).start()
        return 0

    lax.fori_loop(0, _NCH, build_fire, 0, unroll=2)

    t_stage = pltpu.make_async_copy(tt.at[pl.ds(base, _NPT)],
                                    t_v.at[pl.ds(0, _NPT)], sem_t)
    t_stage.start()

    pltpu.make_async_copy(img.at[pl.ds(0, _APAD)],
                          z_v.at[pl.ds(0, _APAD)], sem).wait()
    pltpu.make_async_copy(img.at[pl.ds(0, _APAD)],
                          z_v.at[pl.ds(_APAD, _APAD)], sem).wait()
    t_stage.wait()

    def loss(i, acc):
        acc_log, acc_sq, acc_cnt = acc
        off = pl.multiple_of(i * 16, 16)
        za = z_v[pl.ds(off, 16)]
        zb = z_v[pl.ds(_APAD + off, 16)]
        tf = t_v[pl.ds(off, 16)].astype(jnp.float32)
        pred = za - zb
        e = jnp.exp(-tf * pred)
        ln_y = _softplus_ln(1.0 + e)
        nzf = tf * tf
        p2 = pred * pred
        acc_log = acc_log + nzf * ln_y
        acc_sq = acc_sq + (p2 - nzf * p2)
        acc_cnt = acc_cnt + nzf
        return acc_log, acc_sq, acc_cnt

    z16 = jnp.zeros((16,), jnp.float32)
    acc_log, acc_sq, acc_cnt = lax.fori_loop(0, _NVEC, loss, (z16, z16, z16),
                                             unroll=8)

    out_v[pl.ds(0, 16)] = acc_log
    out_v[pl.ds(16, 16)] = acc_sq
    out_v[pl.ds(32, 16)] = acc_cnt
    pltpu.sync_copy(out_v, parts.at[wid])


@jax.jit
def kernel(output, x_A, y_A, x_B, y_B, ordinal_relation):
    img = (output.reshape(_B, _H // 8, 8, _W // 128, 128)
           .transpose(0, 1, 3, 2, 4).reshape(-1))
    mesh = plsc.VectorSubcoreMesh(core_axis_name="c", subcore_axis_name="s")
    run = functools.partial(
        pl.kernel,
        mesh=mesh,
        out_type=jax.ShapeDtypeStruct((_NW, 48), jnp.float32),
        scratch_types=[
            pltpu.VMEM((_APAD,), jnp.int32),
            pltpu.VMEM((_APAD,), jnp.int32),
            pltpu.VMEM((_APAD,), jnp.int32),
            pltpu.VMEM((_APAD,), jnp.int32),
            pltpu.VMEM((_APAD,), jnp.int32),
            pltpu.VMEM((2 * _APAD,), jnp.int32),
            pltpu.VMEM((2 * _APAD,), jnp.float32),
            pltpu.VMEM((48,), jnp.float32),
            pltpu.SemaphoreType.DMA,
            pltpu.SemaphoreType.DMA,
        ],
    )(_sc_body)
    parts = run(
        img,
        x_A.reshape(-1).astype(jnp.int32),
        y_A.reshape(-1).astype(jnp.int32),
        x_B.reshape(-1).astype(jnp.int32),
        y_B.reshape(-1).astype(jnp.int32),
        ordinal_relation.reshape(-1).astype(jnp.int32),
    )
    p = parts.reshape(_NW, 3, 16).sum(axis=2).reshape(_B, 2, 3).sum(axis=1)
    cnt_nz = p[:, 2]
    log_loss = p[:, 0] / jnp.maximum(cnt_nz, 1.0)
    sq_loss = p[:, 1] / jnp.maximum(_P - cnt_nz, 1.0)
    return jnp.sum(log_loss + sq_loss) / _B


# speedup vs baseline: 1.0085x; 1.0020x over previous
import functools

import jax
import jax.numpy as jnp
from jax import lax
from jax.experimental import pallas as pl
from jax.experimental.pallas import tpu as pltpu
from jax.experimental.pallas import tpu_sc as plsc

_B, _H, _W, _P = 16, 512, 512, 20000
_NW = 32
_NPT = (_B * _P) // _NW
_CHUNK = 16
_NCH = 625
_APAD = _NCH * _CHUNK
_NPAD = (_APAD - _NPT) // 16
_NVEC = _NPT // 16
_LN2 = 0.6931471805599453


def _softplus_ln(y):
    bits = lax.bitcast_convert_type(y, jnp.int32)
    k = (bits >> 23) - 127
    m = lax.bitcast_convert_type((bits & 0x007FFFFF) | 0x3F800000, jnp.float32)
    t = (m - 1.0) / (m + 1.0)
    s = t * t
    ln_m = 2.0 * t * (1.0 + s * (1.0 / 3.0 + s * (1.0 / 5.0 + s * (1.0 / 7.0))))
    return k.astype(jnp.float32) * _LN2 + ln_m


def _sc_body(img, xa, ya, xb, yb, tt, parts,
             xa_v, ya_v, xb_v, yb_v, t_v, z_v, out_v, sem, sem_t):
    wid = lax.axis_index("s") * 2 + lax.axis_index("c")
    base = wid * _NPT
    boff = (wid // 2) * (_H * _W)

    stages = [
        pltpu.make_async_copy(src.at[pl.ds(base, _NPT)],
                              dst.at[pl.ds(0, _NPT)], sem)
        for src, dst in ((xa, xa_v), (ya, ya_v), (xb, xb_v), (yb, yb_v))
    ]
    for h in stages:
        h.start()
    for h in stages:
        h.wait()

    zeros = jnp.zeros((16,), jnp.int32)
    for j in range(_NPAD):
        for ref in (xa_v, ya_v, xb_v, yb_v):
            ref[pl.ds(_NPT + j * 16, 16)] = zeros

    def build_fire(j, _):
        off = pl.multiple_of(j * _CHUNK, _CHUNK)
        for s in range(_CHUNK // 16):
            o = off + s * 16
            xav = xa_v[pl.ds(o, 16)]
            yav = ya_v[pl.ds(o, 16)]
            xbv = xb_v[pl.ds(o, 16)]
            ybv = yb_v[pl.ds(o, 16)]
            iva = ((xav << 9) - (xav & 7) * 384
                   + (yav << 3) - (yav & 127) * 7 + boff)
            ivb = ((xbv << 9) - (xbv & 7) * 384
                   + (ybv << 3) - (ybv & 127) * 7 + boff)
            pltpu.make_async_copy(
                img.at[iva], z_v.at[pl.ds(o, 16)], sem).start()
            pltpu.make_async_copy(
                img.at[ivb], z_v.at[pl.ds(_APAD + o, 16)], sem).start()
        return 0

    lax.fori_loop(0, _NCH, build_fire, 0, unroll=2)

    t_stage = pltpu.make_async_copy(tt.at[pl.ds(base, _NPT)],
                                    t_v.at[pl.ds(0, _NPT)], sem_t)
    t_stage.start()

    pltpu.make_async_copy(img.at[pl.ds(0, _APAD)],
                          z_v.at[pl.ds(0, _APAD)], sem).wait()
    pltpu.make_async_copy(img.at[pl.ds(0, _APAD)],
                          z_v.at[pl.ds(_APAD, _APAD)], sem).wait()
    t_stage.wait()

    def loss(i, acc):
        acc_log, acc_sq, acc_cnt = acc
        off = pl.multiple_of(i * 16, 16)
        za = z_v[pl.ds(off, 16)]
        zb = z_v[pl.ds(_APAD + off, 16)]
        tf = t_v[pl.ds(off, 16)].astype(jnp.float32)
        pred = za - zb
        e = jnp.exp(-tf * pred)
        ln_y = _softplus_ln(1.0 + e)
        nzf = tf * tf
        p2 = pred * pred
        acc_log = acc_log + nzf * ln_y
        acc_sq = acc_sq + (p2 - nzf * p2)
        acc_cnt = acc_cnt + nzf
        return acc_log, acc_sq, acc_cnt

    z16 = jnp.zeros((16,), jnp.float32)
    acc_log, acc_sq, acc_cnt = lax.fori_loop(0, _NVEC, loss, (z16, z16, z16),
                                             unroll=8)

    out_v[pl.ds(0, 16)] = acc_log
    out_v[pl.ds(16, 16)] = acc_sq
    out_v[pl.ds(32, 16)] = acc_cnt
    pltpu.sync_copy(out_v, parts.at[wid])


@jax.jit
def kernel(output, x_A, y_A, x_B, y_B, ordinal_relation):
    img = (output.reshape(_B, _H // 8, 8, _W // 128, 128)
           .transpose(0, 1, 3, 2, 4).reshape(-1))
    mesh = plsc.VectorSubcoreMesh(core_axis_name="c", subcore_axis_name="s")
    run = functools.partial(
        pl.kernel,
        mesh=mesh,
        out_type=jax.ShapeDtypeStruct((_NW, 48), jnp.float32),
        scratch_types=[
            pltpu.VMEM((_APAD,), jnp.int32),
            pltpu.VMEM((_APAD,), jnp.int32),
            pltpu.VMEM((_APAD,), jnp.int32),
            pltpu.VMEM((_APAD,), jnp.int32),
            pltpu.VMEM((_APAD,), jnp.int32),
            pltpu.VMEM((2 * _APAD,), jnp.float32),
            pltpu.VMEM((48,), jnp.float32),
            pltpu.SemaphoreType.DMA,
            pltpu.SemaphoreType.DMA,
        ],
    )(_sc_body)
    parts = run(
        img,
        x_A.reshape(-1).astype(jnp.int32),
        y_A.reshape(-1).astype(jnp.int32),
        x_B.reshape(-1).astype(jnp.int32),
        y_B.reshape(-1).astype(jnp.int32),
        ordinal_relation.reshape(-1).astype(jnp.int32),
    )
    p = parts.reshape(_NW, 3, 16).sum(axis=2).reshape(_B, 2, 3).sum(axis=1)
    cnt_nz = p[:, 2]
    log_loss = p[:, 0] / jnp.maximum(cnt_nz, 1.0)
    sq_loss = p[:, 1] / jnp.maximum(_P - cnt_nz, 1.0)
    return jnp.sum(log_loss + sq_loss) / _B
